# baseline (speedup 1.0000x reference)
import functools

import jax
import jax.numpy as jnp
from jax import lax
from jax.experimental import pallas as pl
from jax.experimental.pallas import tpu as pltpu
from jax.experimental.pallas import tpu_sc as plsc

_N = 20
_P = _N ** 3
_L = 16
_T = 4
_LINES = 400 // _T
_G = -(-_LINES // _L)
_SLAB = _P // _T
_ACC = 4 * _P
_INF = float("inf")


@functools.lru_cache(maxsize=1)
def _build_sc_kernel():
    mesh = plsc.VectorSubcoreMesh(
        core_axis_name="c", subcore_axis_name="s", num_cores=1, num_subcores=16
    )
    return functools.partial(
        pl.kernel,
        out_type=jax.ShapeDtypeStruct((_L,), jnp.float32),
        mesh=mesh,
        scratch_types=[
            pltpu.VMEM((_P,), jnp.float32),
            pltpu.VMEM((_P,), jnp.float32),
            pltpu.VMEM((_P,), jnp.float32),
            pltpu.VMEM((_P,), jnp.float32),
            pltpu.VMEM((_L,), jnp.float32),
            pltpu.VMEM((16 * _L,), jnp.float32),
            pltpu.VMEM((_L,), jnp.float32),
            pltpu.VMEM_SHARED((_ACC + 16 * _L,), jnp.float32),
            pltpu.SemaphoreType.DMA,
            pltpu.SemaphoreType.DMA,
        ],
        compiler_params=pltpu.CompilerParams(needs_layout_passes=False),
    )(_hausdorff_sc_body)


def _hausdorff_sc_body(pred_hbm, targ_hbm, out_hbm,
                       src_v, oth_v, da_v, db_v, acc_v, part_v, res_v,
                       shared, sem_s, sem_o):
    s = lax.axis_index("s")
    vol = s // _T
    t = s % _T
    batch = vol // 2
    is_a = (vol % 2) == 0
    lane = lax.iota(jnp.int32, _L)
    w = [[float((z - zp) ** 2) for z in range(_N)] for zp in range(_N)]

    def run_pass(load_in, handle_out, base_of, stride):
        def group_body(g, carry):
            line = jnp.minimum(t * _LINES + g * _L + lane, 399)
            base = base_of(line)
            ins = [load_in(base + stride * zp) for zp in range(_N)]
            for z in range(_N):
                o = ins[z]
                for zp in range(_N):
                    if zp != z:
                        o = jnp.minimum(o, ins[zp] + w[zp][z])
                handle_out(base + stride * z, o)
            return carry
        lax.fori_loop(0, _G, group_body, 0)

    @pl.when(is_a)
    def _():
        pltpu.async_copy(targ_hbm.at[batch], src_v, sem_s)
        pltpu.async_copy(pred_hbm.at[batch], oth_v, sem_o)

    @pl.when(jnp.logical_not(is_a))
    def _():
        pltpu.async_copy(pred_hbm.at[batch], src_v, sem_s)
        pltpu.async_copy(targ_hbm.at[batch], oth_v, sem_o)

    pltpu.make_async_copy(pred_hbm.at[batch], src_v, sem_s).wait()

    def load_z(idx):
        return jnp.where(plsc.load_gather(src_v, [idx]) > 0.5, 0.0, _INF)

    run_pass(load_z,
             lambda idx, o: plsc.store_scatter(da_v, [idx], o),
             lambda l: l * _N, 1)

    run_pass(lambda idx: plsc.load_gather(da_v, [idx]),
             lambda idx, o: plsc.store_scatter(db_v, [idx], o),
             lambda l: (l // _N) * (_N * _N) + (l % _N), _N)

    pltpu.sync_copy(db_v.at[pl.ds(t * _SLAB, _SLAB)],
                    shared.at[pl.ds(vol * _P + t * _SLAB, _SLAB)])

    plsc.subcore_barrier()

    pltpu.make_async_copy(pred_hbm.at[batch], oth_v, sem_o).wait()
    pltpu.sync_copy(shared.at[pl.ds(vol * _P, _P)], da_v)
    acc_v[...] = jnp.full((_L,), -1.0, jnp.float32)

    def reduce_x(idx, o):
        qs = plsc.load_gather(src_v, [idx])
        qo = plsc.load_gather(oth_v, [idx])
        q = (qo > 0.5) & jnp.logical_not(qs > 0.5)
        acc_v[...] = jnp.maximum(acc_v[...], jnp.where(q, o, -1.0))

    run_pass(lambda idx: plsc.load_gather(da_v, [idx]),
             reduce_x, lambda l: l, _N * _N)

    pltpu.sync_copy(acc_v, shared.at[pl.ds(_ACC + _L * s, _L)])

    plsc.subcore_barrier()

    @pl.when(s == 0)
    def _combine():
        pltpu.sync_copy(shared.at[pl.ds(_ACC, 16 * _L)], part_v)
        mqs = []
        for v in range(4):
            m = part_v[pl.ds(_L * (_T * v), _L)]
            for k in range(1, _T):
                m = jnp.maximum(m, part_v[pl.ds(_L * (_T * v + k), _L)])
            mqs.append(jnp.max(m))
        mq = jnp.where(lane == 0, mqs[0],
                       jnp.where(lane == 1, mqs[1],
                                 jnp.where(lane == 2, mqs[2], mqs[3])))
        x = jnp.maximum(mq, 0.0)
        seed = plsc.bitcast(
            (lax.shift_right_logical(plsc.bitcast(x, jnp.int32), 1)
             + jnp.int32(0x1FBD1DF6)), jnp.float32)
        y = seed
        for _ in range(3):
            y = 0.5 * (y + x / y)
        d = y * jnp.float32(1.0 / _N)
        da = jnp.where(mq < 0.0, 0.0, jnp.where(mq > 1e9, _INF, d))
        db = jnp.where(mq < 0.0, 0.0, jnp.where(mq > 1e9, 999.0, d))
        dist = jnp.where(lane % 2 == 1, db, da)
        h0 = jnp.max(jnp.where(lane < 2, dist, -_INF))
        h1 = jnp.max(jnp.where((lane >= 2) & (lane < 4), dist, -_INF))
        res_v[...] = jnp.where(lane == 0, 0.5 * (h0 + h1), 0.0)
        pltpu.sync_copy(res_v, out_hbm)


def kernel(predict, target):
    pred = predict.reshape(2, _P)
    targ = target.reshape(2, _P)
    out = _build_sc_kernel()(pred, targ)
    return out[0]


# speedup vs baseline: 1.0219x; 1.0219x over previous
import functools

import jax
import jax.numpy as jnp
from jax import lax
from jax.experimental import pallas as pl
from jax.experimental.pallas import tpu as pltpu
from jax.experimental.pallas import tpu_sc as plsc

_N = 20
_P = _N ** 3
_W = _P // 2
_L = 16
_T = 4
_GZ = -(-(400 // _T) // _L)
_GP = -(-(200 // _T) // _L)
_SLABW = _W // _T
_S = 30000
_INF = float("inf")


@functools.lru_cache(maxsize=1)
def _build_sc_kernel():
    mesh = plsc.VectorSubcoreMesh(
        core_axis_name="c", subcore_axis_name="s", num_cores=1, num_subcores=16
    )
    return functools.partial(
        pl.kernel,
        out_type=jax.ShapeDtypeStruct((_L,), jnp.float32),
        mesh=mesh,
        scratch_types=[
            pltpu.VMEM((_P,), jnp.float32),
            pltpu.VMEM((_P,), jnp.float32),
            pltpu.VMEM((_W,), jnp.int32),
            pltpu.VMEM((_W,), jnp.int32),
            pltpu.VMEM((2 * _L,), jnp.int16),
            pltpu.VMEM((16 * _L,), jnp.float32),
            pltpu.VMEM((_L,), jnp.float32),
            pltpu.VMEM_SHARED((4 * _W,), jnp.int32),
            pltpu.VMEM_SHARED((16 * _L,), jnp.float32),
            pltpu.SemaphoreType.DMA,
            pltpu.SemaphoreType.DMA,
        ],
        compiler_params=pltpu.CompilerParams(needs_layout_passes=False),
    )(_hausdorff_sc_body)


def _i16(x32):
    return plsc.bitcast(x32, jnp.int16)


def _i32(x16):
    return plsc.bitcast(x16, jnp.int32)


def _hausdorff_sc_body(pred_hbm, targ_hbm, out_hbm,
                       src_v, oth_v, da_v, db_v, acc_v, part_v, res_v,
                       sh_d, sh_acc, sem_s, sem_o):
    s = lax.axis_index("s")
    vol = s // _T
    t = s % _T
    batch = vol // 2
    is_a = (vol % 2) == 0
    lane = lax.iota(jnp.int32, _L)
    w = [[(z - zp) ** 2 for z in range(_N)] for zp in range(_N)]

    @pl.when(is_a)
    def _():
        pltpu.async_copy(targ_hbm.at[batch], src_v, sem_s)
        pltpu.async_copy(pred_hbm.at[batch], oth_v, sem_o)

    @pl.when(jnp.logical_not(is_a))
    def _():
        pltpu.async_copy(pred_hbm.at[batch], src_v, sem_s)
        pltpu.async_copy(targ_hbm.at[batch], oth_v, sem_o)

    pltpu.make_async_copy(pred_hbm.at[batch], src_v, sem_s).wait()

    def z_group(g, carry):
        line = jnp.minimum(t * (400 // _T) + g * _L + lane, 399)
        dup = [jnp.where(plsc.load_gather(src_v, [line * _N + zp]) > 0.5,
                         0, _S) * 65537
               for zp in range(_N)]
        for zpair in range(_N // 2):
            wp = [w[zp][2 * zpair] | (w[zp][2 * zpair + 1] << 16)
                  for zp in range(_N)]
            o = _i16(dup[0] + wp[0])
            for zp in range(1, _N):
                o = jnp.minimum(o, _i16(dup[zp] + wp[zp]))
            plsc.store_scatter(da_v, [line * (_N // 2) + zpair], _i32(o))
        return carry

    lax.fori_loop(0, _GZ, z_group, 0)

    def y_group(g, carry):
        l2 = jnp.minimum(t * (200 // _T) + g * _L + lane, 199)
        base = (l2 // (_N // 2)) * (_N * _N // 2) + (l2 % (_N // 2))
        ins = [plsc.load_gather(da_v, [base + (_N // 2) * yp])
               for yp in range(_N)]
        for y in range(_N):
            o = _i16(ins[0] + w[0][y] * 65537)
            for yp in range(1, _N):
                o = jnp.minimum(o, _i16(ins[yp] + w[yp][y] * 65537))
            plsc.store_scatter(db_v, [base + (_N // 2) * y], _i32(o))
        return carry

    lax.fori_loop(0, _GP, y_group, 0)

    pltpu.sync_copy(db_v.at[pl.ds(t * _SLABW, _SLABW)],
                    sh_d.at[pl.ds(vol * _W + t * _SLABW, _SLABW)])

    plsc.subcore_barrier()

    pltpu.make_async_copy(pred_hbm.at[batch], oth_v, sem_o).wait()
    pltpu.sync_copy(sh_d.at[pl.ds(vol * _W, _W)], da_v)
    acc_v[...] = jnp.full((2 * _L,), -1, jnp.int16)

    def x_group(g, carry):
        l2 = jnp.minimum(t * (200 // _T) + g * _L + lane, 199)
        ins = [plsc.load_gather(da_v, [l2 + (_N * _N // 2) * xp])
               for xp in range(_N)]
        for x in range(_N):
            o = _i16(ins[0] + w[0][x] * 65537)
            for xp in range(1, _N):
                o = jnp.minimum(o, _i16(ins[xp] + w[xp][x] * 65537))
            vlo = x * (_N * _N) + 2 * l2
            qm = jnp.int32(0)
            for half in range(2):
                qs = plsc.load_gather(src_v, [vlo + half])
                qo = plsc.load_gather(oth_v, [vlo + half])
                q = (qo > 0.5) & jnp.logical_not(qs > 0.5)
                qm = qm | (jnp.where(q, 1, 0) << (16 * half))
            acc_v[...] = jnp.maximum(
                acc_v[...],
                jnp.where(_i16(qm) > 0, o, jnp.int16(-1)))
        return carry

    lax.fori_loop(0, _GP, x_group, 0)

    word = _i32(acc_v[...])
    halves = jnp.maximum((word << 16) >> 16, word >> 16)
    res_v[...] = halves.astype(jnp.float32)
    pltpu.sync_copy(res_v, sh_acc.at[pl.ds(_L * s, _L)])

    plsc.subcore_barrier()

    @pl.when(s == 0)
    def _combine():
        pltpu.sync_copy(sh_acc, part_v)
        mqs = []
        for v in range(4):
            m = part_v[pl.ds(_L * (_T * v), _L)]
            for k in range(1, _T):
                m = jnp.maximum(m, part_v[pl.ds(_L * (_T * v + k), _L)])
            mqs.append(jnp.max(m))
        mq = jnp.where(lane == 0, mqs[0],
                       jnp.where(lane == 1, mqs[1],
                                 jnp.where(lane == 2, mqs[2], mqs[3])))
        x = jnp.maximum(mq, 0.0)
        seed = plsc.bitcast(
            (lax.shift_right_logical(plsc.bitcast(x, jnp.int32), 1)
             + jnp.int32(0x1FBD1DF6)), jnp.float32)
        y = seed
        for _ in range(3):
            y = 0.5 * (y + x / y)
        d = y * jnp.float32(1.0 / _N)
        empty_src = mq > jnp.float32(_S - 1000)
        da = jnp.where(mq < 0.0, 0.0, jnp.where(empty_src, _INF, d))
        db = jnp.where(mq < 0.0, 0.0, jnp.where(empty_src, 999.0, d))
        dist = jnp.where(lane % 2 == 1, db, da)
        h0 = jnp.max(jnp.where(lane < 2, dist, -_INF))
        h1 = jnp.max(jnp.where((lane >= 2) & (lane < 4), dist, -_INF))
        res_v[...] = jnp.where(lane == 0, 0.5 * (h0 + h1), 0.0)
        pltpu.sync_copy(res_v, out_hbm)


def kernel(predict, target):
    pred = predict.reshape(2, _P)
    targ = target.reshape(2, _P)
    out = _build_sc_kernel()(pred, targ)
    return out[0]
